# Initial kernel scaffold; baseline (speedup 1.0000x reference)
import jax
import jax.numpy as jnp
from jax.experimental import pallas as pl


def kernel(X, labels_cam):
    raise NotImplementedError("write your pallas kernel here")


# speedup vs baseline: 9.6056x; 9.6056x over previous
import jax
import jax.numpy as jnp
from jax.experimental import pallas as pl

_N = 4096
_D = 128
_K1 = 30
_K2 = 6
_INV_B1 = 5.0
_INV_B2 = 5.0
_SCALE = 0.3
_BLK = 256
_GRID = _N // _BLK


def _keys_of(s):
    b = jax.lax.bitcast_convert_type(s, jnp.int32)
    return b ^ ((b >> 31) & jnp.int32(0x7FFFFFFF))


def _count_ge(keys, t, ones_col):
    ind = jnp.where(keys >= t, 1.0, 0.0)
    return jax.lax.dot_general(ind, ones_col, (((1,), (0,)), ((), ())),
                               preferred_element_type=jnp.float32)


def _kth_threshold(keys, k, ones_col):
    kf = jnp.float32(k)
    cnt0 = _count_ge(keys, jnp.int32(0), ones_col)
    t0 = jnp.where(cnt0 >= kf, jnp.int32(0), jnp.int32(-2147483648))

    def step(i, t):
        bit = jax.lax.shift_left(jnp.int32(1), jnp.int32(30) - i)
        tt = t + bit
        cnt = _count_ge(keys, tt, ones_col)
        return jnp.where(cnt >= kf, tt, t)

    return jax.lax.fori_loop(0, 31, step, t0)


def _sim_blocks(i, Xb_ref, Xall_ref, labr_ref, labc_ref):
    sim = jax.lax.dot_general(
        Xb_ref[...], Xall_ref[...], (((1,), (1,)), ((), ())),
        preferred_element_type=jnp.float32)
    col_ids = jax.lax.broadcasted_iota(jnp.int32, sim.shape, 1)
    row_ids = jax.lax.broadcasted_iota(jnp.int32, sim.shape, 0) + i * _BLK
    eye = col_ids == row_ids
    suppress = (labr_ref[...] == labc_ref[...]) & (~eye)
    sim2 = jnp.where(suppress, jnp.float32(-2.0), sim)
    return sim, sim2, eye


def _pass_a(Xb_ref, Xall_ref, labr_ref, labc_ref,
            t1_ref, t2_ref, dr1_ref, dr2_ref, c1_ref, c2_ref):
    i = pl.program_id(0)
    sim, sim2, eye = _sim_blocks(i, Xb_ref, Xall_ref, labr_ref, labc_ref)
    ones_col = jnp.ones((_N, 1), jnp.float32)

    k1 = _keys_of(sim)
    t1 = _kth_threshold(k1, _K1, ones_col)
    mask1 = k1 >= t1
    k2 = _keys_of(sim2)
    t2 = _kth_threshold(k2, _K2, ones_col)
    mask2 = (k2 >= t2) | eye

    S1 = jnp.where(mask1, jnp.exp(sim * _INV_B1), 0.0)
    S2 = jnp.where(mask2, jnp.exp(sim2 * _INV_B2), 0.0)

    t1_ref[...] = t1
    t2_ref[...] = t2
    dr1_ref[...] = jax.lax.rsqrt(jnp.sum(S1, axis=1, keepdims=True))
    dr2_ref[...] = jax.lax.rsqrt(jnp.sum(S2, axis=1, keepdims=True))

    c1 = jnp.sum(S1, axis=0, keepdims=True)
    c2 = jnp.sum(S2, axis=0, keepdims=True)

    @pl.when(i == 0)
    def _():
        c1_ref[...] = c1
        c2_ref[...] = c2

    @pl.when(i > 0)
    def _():
        c1_ref[...] += c1
        c2_ref[...] += c2


def _pass_b(Xb_ref, Xall_ref, labr_ref, labc_ref,
            t1_ref, t2_ref, dr1_ref, dr2_ref, c1t_ref, c2t_ref, out_ref):
    i = pl.program_id(0)
    sim, sim2, eye = _sim_blocks(i, Xb_ref, Xall_ref, labr_ref, labc_ref)

    mask1 = _keys_of(sim) >= t1_ref[...]
    mask2 = (_keys_of(sim2) >= t2_ref[...]) | eye
    S1 = jnp.where(mask1, jnp.exp(sim * _INV_B1), 0.0)
    S2 = jnp.where(mask2, jnp.exp(sim2 * _INV_B2), 0.0)

    W1 = jax.lax.rsqrt(c1t_ref[...]) * Xall_ref[...]
    W2 = jax.lax.rsqrt(c2t_ref[...]) * Xall_ref[...]
    g = jax.lax.dot_general(S1, W1, (((1,), (0,)), ((), ())),
                            preferred_element_type=jnp.float32)
    c = jax.lax.dot_general(S2, W2, (((1,), (0,)), ((), ())),
                            preferred_element_type=jnp.float32)

    Xo = _SCALE * (dr2_ref[...] * c) + (1.0 - _SCALE) * (dr1_ref[...] * g)
    out_ref[...] = Xo * jax.lax.rsqrt(jnp.sum(Xo * Xo, axis=1, keepdims=True))


def kernel(X, labels_cam):
    labr = labels_cam.reshape(_N, 1)
    labc = labels_cam.reshape(1, _N)

    row_spec = pl.BlockSpec((_BLK, 1), lambda i: (i, 0))
    full_x_spec = pl.BlockSpec((_N, _D), lambda i: (0, 0))
    col_spec = pl.BlockSpec((1, _N), lambda i: (0, 0))

    t1, t2, dr1, dr2, c1, c2 = pl.pallas_call(
        _pass_a,
        grid=(_GRID,),
        in_specs=[
            pl.BlockSpec((_BLK, _D), lambda i: (i, 0)),
            full_x_spec,
            row_spec,
            col_spec,
        ],
        out_specs=[row_spec, row_spec, row_spec, row_spec, col_spec, col_spec],
        out_shape=[
            jax.ShapeDtypeStruct((_N, 1), jnp.int32),
            jax.ShapeDtypeStruct((_N, 1), jnp.int32),
            jax.ShapeDtypeStruct((_N, 1), jnp.float32),
            jax.ShapeDtypeStruct((_N, 1), jnp.float32),
            jax.ShapeDtypeStruct((1, _N), jnp.float32),
            jax.ShapeDtypeStruct((1, _N), jnp.float32),
        ],
    )(X, X, labr, labc)

    c1t = c1.reshape(_N, 1)
    c2t = c2.reshape(_N, 1)
    col_t_spec = pl.BlockSpec((_N, 1), lambda i: (0, 0))

    Xo = pl.pallas_call(
        _pass_b,
        grid=(_GRID,),
        in_specs=[
            pl.BlockSpec((_BLK, _D), lambda i: (i, 0)),
            full_x_spec,
            row_spec,
            col_spec,
            row_spec,
            row_spec,
            row_spec,
            row_spec,
            col_t_spec,
            col_t_spec,
        ],
        out_specs=pl.BlockSpec((_BLK, _D), lambda i: (i, 0)),
        out_shape=jax.ShapeDtypeStruct((_N, _D), jnp.float32),
    )(X, X, labr, labc, t1, t2, dr1, dr2, c1t, c2t)
    return Xo
